# baseline (speedup 1.0000x reference)
import functools

import jax
import jax.numpy as jnp
from jax import lax
from jax.experimental import pallas as pl
from jax.experimental.pallas import tpu as pltpu
from jax.experimental.pallas import tpu_sc as plsc

_B, _N = 4096, 8192
_K = (_N - 1) // 2
_NW = 32
_RPW = _B // _NW
_NV = _N // 16
_IMIN = -(2 ** 31)
_IMAX = 2 ** 31 - 1


def _sc_body(x_hbm, val_hbm, idx_hbm, xbuf, keys, hist, cand, candidx,
             valbuf, idxbuf):
    nc = 2
    wid = lax.axis_index("s") * nc + lax.axis_index("c")
    lane = lax.iota(jnp.int32, 16)
    ones = jnp.ones((16,), jnp.int32)
    zeros16 = jnp.zeros((16,), jnp.int32)
    lane0 = lane == 0

    def splat(s):
        return jnp.broadcast_to(s, (16,))

    def row_body(r, _):
        g = wid * _RPW + r
        pltpu.sync_copy(x_hbm.at[g], xbuf)

        def zero_body(i, _c):
            hist[pl.ds(pl.multiple_of(i * 16, 8), 16)] = zeros16
            return 0

        lax.fori_loop(0, 256, zero_body, 0)

        def keys_body(j, _c):
            off = pl.multiple_of(j * 16, 8)
            v = xbuf[pl.ds(off, 16)]
            v = jnp.where(v == 0.0, jnp.float32(0.0), v)
            v = jnp.where(v != v, jnp.float32(jnp.nan), v)
            bits = lax.bitcast_convert_type(v, jnp.int32)
            key = bits ^ ((bits >> 31) & jnp.int32(0x7FFFFFFF))
            keys[pl.ds(off, 16)] = key
            dig = ((key >> 24) & 255) ^ 128
            plsc.addupdate_scatter(hist, [dig * 16 + lane], ones)
            return 0

        lax.fori_loop(0, _NV, keys_body, 0)

        def scan_body(i, carry):
            cum, bstar, below = carry
            h = hist[pl.ds(pl.multiple_of(i * 16, 8), 16)]
            s = jnp.sum(h)
            newcum = cum + s
            hit = (cum <= _K) & (newcum > _K)
            bstar = jnp.where(hit, i, bstar)
            below = jnp.where(hit, cum, below)
            return newcum, bstar, below

        _, bstar, below = lax.fori_loop(
            0, 256, scan_body,
            (jnp.int32(0), jnp.int32(0), jnp.int32(0)))

        def compact_body(j, offv):
            off8 = pl.multiple_of(j * 16, 8)
            key = keys[pl.ds(off8, 16)]
            dig = ((key >> 24) & 255) ^ 128
            m = dig == splat(bstar)
            mi = m.astype(jnp.int32)
            pos = offv + plsc.cumsum(mi) - mi
            plsc.store_scatter(cand, [pos], key ^ jnp.int32(_IMIN), mask=m)
            plsc.store_scatter(candidx, [pos], off8 + lane, mask=m)
            return offv + plsc.all_reduce_population_count(m)

        offv = lax.fori_loop(0, _NV, compact_body, zeros16)
        csize = jnp.max(offv)
        plsc.store_scatter(cand, [offv + lane], splat(jnp.int32(_IMAX)))
        plsc.store_scatter(candidx, [offv + lane], splat(jnp.int32(1 << 20)))
        nv = (csize + 15) // 16

        t_c = jnp.int32(_K) - below

        def count_lt(trial):
            def cbody(v, cnt):
                kv = cand[pl.ds(pl.multiple_of(v * 16, 8), 16)]
                return cnt + jnp.sum((kv < splat(trial)).astype(jnp.int32))
            return lax.fori_loop(0, nv, cbody, jnp.int32(0))

        lo = lax.shift_left(bstar, 24)
        for b in range(23, -1, -1):
            trial = lo | jnp.int32(1 << b)
            lo = jnp.where(count_lt(trial) <= t_c, trial, lo)
        uvkey = lo
        t = t_c - count_lt(uvkey)

        def count_idx(trial):
            def cbody(v, cnt):
                off8 = pl.multiple_of(v * 16, 8)
                kv = cand[pl.ds(off8, 16)]
                iv = candidx[pl.ds(off8, 16)]
                m = (kv == splat(uvkey)) & (iv < splat(trial))
                return cnt + jnp.sum(m.astype(jnp.int32))
            return lax.fori_loop(0, nv, cbody, jnp.int32(0))

        ilo = jnp.int32(0)
        for b in range(12, -1, -1):
            trial = ilo | jnp.int32(1 << b)
            ilo = jnp.where(count_idx(trial) <= t, trial, ilo)

        vkey = uvkey ^ jnp.int32(_IMIN)
        vbits = vkey ^ ((vkey >> 31) & jnp.int32(0x7FFFFFFF))
        vvec = lax.bitcast_convert_type(splat(vbits), jnp.float32)
        plsc.store_scatter(valbuf, [splat(r)], vvec, mask=lane0)
        plsc.store_scatter(idxbuf, [splat(r)], splat(ilo), mask=lane0)
        return 0

    lax.fori_loop(0, _RPW, row_body, 0)
    base = wid * _RPW
    pltpu.sync_copy(valbuf, val_hbm.at[pl.ds(base, _RPW)])
    pltpu.sync_copy(idxbuf, idx_hbm.at[pl.ds(base, _RPW)])


@jax.jit
def kernel(x):
    mesh = plsc.VectorSubcoreMesh(core_axis_name="c", subcore_axis_name="s",
                              num_cores=2, num_subcores=16)
    f = pl.kernel(
        _sc_body,
        out_type=[
            jax.ShapeDtypeStruct((_B,), jnp.float32),
            jax.ShapeDtypeStruct((_B,), jnp.int32),
        ],
        mesh=mesh,
        compiler_params=pltpu.CompilerParams(needs_layout_passes=False),
        scratch_types=[
            pltpu.VMEM((_N,), jnp.float32),
            pltpu.VMEM((_N,), jnp.int32),
            pltpu.VMEM((4096,), jnp.int32),
            pltpu.VMEM((_N + 16,), jnp.int32),
            pltpu.VMEM((_N + 16,), jnp.int32),
            pltpu.VMEM((_RPW,), jnp.float32),
            pltpu.VMEM((_RPW,), jnp.int32),
        ],
    )
    values, idx = f(x)
    return values, idx


# speedup vs baseline: 2.8680x; 2.8680x over previous
import jax
import jax.numpy as jnp
from jax.experimental import pallas as pl

_B, _N = 4096, 8192
_K = (_N - 1) // 2
_BR = 128
_IMIN = -(2 ** 31)


def _sbit(b: int) -> int:
    v = 1 << b
    return v - (1 << 32) if v >= (1 << 31) else v


def _median_block(x_ref, val_ref, idx_ref):
    x = x_ref[...]
    x = jnp.where(x == 0.0, jnp.float32(0.0), x)
    x = jnp.where(jnp.isnan(x), jnp.float32(jnp.nan), x)
    bits = jax.lax.bitcast_convert_type(x, jnp.int32)
    skey = bits ^ ((bits >> 31) & 0x7FFFFFFF)

    lo = jnp.zeros((_BR, 1), jnp.int32)
    kk = jnp.int32(_K)
    for b in range(31, -1, -1):
        trial_u = lo | jnp.int32(_sbit(b))
        trial_s = trial_u ^ jnp.int32(_IMIN)
        cnt = jnp.sum((skey < trial_s).astype(jnp.int32), axis=1,
                      keepdims=True)
        lo = jnp.where(cnt <= kk, trial_u, lo)

    vkey = lo ^ jnp.int32(_IMIN)
    cnt_less = jnp.sum((skey < vkey).astype(jnp.int32), axis=1,
                       keepdims=True)
    t = kk - cnt_less

    iot = jax.lax.broadcasted_iota(jnp.int32, (_BR, _N), 1)
    w = jnp.where(skey == vkey, iot, jnp.int32(_N))
    ilo = jnp.zeros((_BR, 1), jnp.int32)
    for b in range(12, -1, -1):
        trial = ilo | jnp.int32(1 << b)
        cnt = jnp.sum((w < trial).astype(jnp.int32), axis=1, keepdims=True)
        ilo = jnp.where(cnt <= t, trial, ilo)

    vbits = vkey ^ ((vkey >> 31) & 0x7FFFFFFF)
    val_ref[...] = jax.lax.bitcast_convert_type(vbits, jnp.float32)
    idx_ref[...] = ilo


@jax.jit
def kernel(x):
    grid = (_B // _BR,)
    values, idx = pl.pallas_call(
        _median_block,
        grid=grid,
        in_specs=[pl.BlockSpec((_BR, _N), lambda i: (i, 0))],
        out_specs=[
            pl.BlockSpec((_BR, 1), lambda i: (i, 0)),
            pl.BlockSpec((_BR, 1), lambda i: (i, 0)),
        ],
        out_shape=[
            jax.ShapeDtypeStruct((_B, 1), jnp.float32),
            jax.ShapeDtypeStruct((_B, 1), jnp.int32),
        ],
    )(x)
    return values[:, 0], idx[:, 0]


# speedup vs baseline: 5.0161x; 1.7490x over previous
import jax
import jax.numpy as jnp
from jax.experimental import pallas as pl

_B, _N = 4096, 8192
_K = (_N - 1) // 2
_BR = 256


def _median_block(x_ref, val_ref, idx_ref):
    x = x_ref[...]
    x = jnp.where(x == 0.0, jnp.float32(0.0), x)
    x = jnp.where(jnp.isnan(x), jnp.float32(jnp.nan), x)
    bits = jax.lax.bitcast_convert_type(x, jnp.int32)
    skey = bits ^ ((bits >> 31) & 0x7FFFFFFF)

    hi = (skey >> 16).astype(jnp.int16)
    lo = ((skey & 0xFFFF) - 32768).astype(jnp.int16)
    kk = jnp.int32(_K)

    def cnt16(mask16):
        m = mask16.astype(jnp.int16)
        while m.shape[1] > 512:
            h = m.shape[1] // 2
            m = m[:, :h] + m[:, h:]
        return jnp.sum(m.astype(jnp.int32), axis=1, keepdims=True)

    hu = jnp.zeros((_BR, 1), jnp.int32)
    for b in range(15, -1, -1):
        trial = ((hu | (1 << b)) - 32768).astype(jnp.int16)
        hu = jnp.where(cnt16(hi < trial) <= kk, hu | (1 << b), hu)
    hs = (hu - 32768).astype(jnp.int16)
    c_hi = cnt16(hi < hs)

    w2 = jnp.where(hi == hs, lo, jnp.int16(32767))
    lu = jnp.zeros((_BR, 1), jnp.int32)
    for b in range(15, -1, -1):
        trial = ((lu | (1 << b)) - 32768).astype(jnp.int16)
        lu = jnp.where(c_hi + cnt16(w2 < trial) <= kk, lu | (1 << b), lu)
    ls = (lu - 32768).astype(jnp.int16)
    t = kk - (c_hi + cnt16(w2 < ls))

    iot = jax.lax.broadcasted_iota(jnp.int32, (_BR, _N), 1).astype(jnp.int16)
    w3 = jnp.where((hi == hs) & (lo == ls), iot, jnp.int16(_N))
    ilo = jnp.zeros((_BR, 1), jnp.int32)
    for b in range(12, -1, -1):
        trial = (ilo | (1 << b)).astype(jnp.int16)
        ilo = jnp.where(cnt16(w3 < trial) <= t, ilo | (1 << b), ilo)

    vkey = (hs.astype(jnp.int32) << 16) | lu
    vbits = vkey ^ ((vkey >> 31) & 0x7FFFFFFF)
    val_ref[...] = jax.lax.bitcast_convert_type(vbits, jnp.float32)
    idx_ref[...] = ilo


@jax.jit
def kernel(x):
    grid = (_B // _BR,)
    values, idx = pl.pallas_call(
        _median_block,
        grid=grid,
        in_specs=[pl.BlockSpec((_BR, _N), lambda i: (i, 0))],
        out_specs=[
            pl.BlockSpec((_BR, 1), lambda i: (i, 0)),
            pl.BlockSpec((_BR, 1), lambda i: (i, 0)),
        ],
        out_shape=[
            jax.ShapeDtypeStruct((_B, 1), jnp.float32),
            jax.ShapeDtypeStruct((_B, 1), jnp.int32),
        ],
    )(x)
    return values[:, 0], idx[:, 0]


# speedup vs baseline: 5.4885x; 1.0942x over previous
import jax
import jax.numpy as jnp
from jax import lax
from jax.experimental import pallas as pl
from jax.experimental.pallas import tpu as pltpu
from jax.experimental.pallas import tpu_sc as plsc

_B, _N = 4096, 8192
_K = (_N - 1) // 2
_SPLIT = 3584
_BR = 256
_NW = 32
_RPW = (_B - _SPLIT) // _NW
_NV = _N // 16
_IMIN = -(2 ** 31)
_IMAX = 2 ** 31 - 1


def _tc_block(x_ref, val_ref, idx_ref):
    x = x_ref[...]
    x = jnp.where(x == 0.0, jnp.float32(0.0), x)
    x = jnp.where(jnp.isnan(x), jnp.float32(jnp.nan), x)
    bits = jax.lax.bitcast_convert_type(x, jnp.int32)
    skey = bits ^ ((bits >> 31) & 0x7FFFFFFF)

    hi = (skey >> 16).astype(jnp.int16)
    lo = ((skey & 0xFFFF) - 32768).astype(jnp.int16)
    kk = jnp.int32(_K)

    def cnt16(mask16):
        m = mask16.astype(jnp.int16)
        while m.shape[1] > 512:
            h = m.shape[1] // 2
            m = m[:, :h] + m[:, h:]
        return jnp.sum(m.astype(jnp.int32), axis=1, keepdims=True)

    hu = jnp.zeros((_BR, 1), jnp.int32)
    for b in range(15, -1, -1):
        trial = ((hu | (1 << b)) - 32768).astype(jnp.int16)
        hu = jnp.where(cnt16(hi < trial) <= kk, hu | (1 << b), hu)
    hs = (hu - 32768).astype(jnp.int16)
    c_hi = cnt16(hi < hs)

    w2 = jnp.where(hi == hs, lo, jnp.int16(32767))
    lu = jnp.zeros((_BR, 1), jnp.int32)
    for b in range(15, -1, -1):
        trial = ((lu | (1 << b)) - 32768).astype(jnp.int16)
        lu = jnp.where(c_hi + cnt16(w2 < trial) <= kk, lu | (1 << b), lu)
    ls = (lu - 32768).astype(jnp.int16)
    t = kk - (c_hi + cnt16(w2 < ls))

    iot = jax.lax.broadcasted_iota(jnp.int32, (_BR, _N), 1).astype(jnp.int16)
    w3 = jnp.where((hi == hs) & (lo == ls), iot, jnp.int16(_N))
    ilo = jnp.zeros((_BR, 1), jnp.int32)
    for b in range(12, -1, -1):
        trial = (ilo | (1 << b)).astype(jnp.int16)
        ilo = jnp.where(cnt16(w3 < trial) <= t, ilo | (1 << b), ilo)

    vkey = (hs.astype(jnp.int32) << 16) | lu
    vbits = vkey ^ ((vkey >> 31) & 0x7FFFFFFF)
    val_ref[...] = jax.lax.bitcast_convert_type(vbits, jnp.float32)
    idx_ref[...] = ilo


def _tc_call(x):
    values, idx = pl.pallas_call(
        _tc_block,
        grid=(_SPLIT // _BR,),
        in_specs=[pl.BlockSpec((_BR, _N), lambda i: (i, 0))],
        out_specs=[
            pl.BlockSpec((_BR, 1), lambda i: (i, 0)),
            pl.BlockSpec((_BR, 1), lambda i: (i, 0)),
        ],
        out_shape=[
            jax.ShapeDtypeStruct((_SPLIT, 1), jnp.float32),
            jax.ShapeDtypeStruct((_SPLIT, 1), jnp.int32),
        ],
    )(x)
    return values[:, 0], idx[:, 0]


def _sc_body(x_hbm, val_hbm, idx_hbm, xbuf, keys, hist, cand, candidx,
             valbuf, idxbuf):
    nc = 2
    wid = lax.axis_index("s") * nc + lax.axis_index("c")
    lane = lax.iota(jnp.int32, 16)
    ones = jnp.ones((16,), jnp.int32)
    zeros16 = jnp.zeros((16,), jnp.int32)
    lane0 = lane == 0

    def splat(s):
        return jnp.broadcast_to(s, (16,))

    def row_body(r, _):
        g = _SPLIT + wid * _RPW + r
        pltpu.sync_copy(x_hbm.at[g], xbuf)

        def zero_body(i, _c):
            hist[pl.ds(pl.multiple_of(i * 16, 8), 16)] = zeros16
            return 0

        lax.fori_loop(0, 256, zero_body, 0)

        def keys_body(j, _c):
            off = pl.multiple_of(j * 16, 8)
            v = xbuf[pl.ds(off, 16)]
            v = jnp.where(v == 0.0, jnp.float32(0.0), v)
            v = jnp.where(v != v, jnp.float32(jnp.nan), v)
            bits = lax.bitcast_convert_type(v, jnp.int32)
            key = bits ^ ((bits >> 31) & jnp.int32(0x7FFFFFFF))
            keys[pl.ds(off, 16)] = key
            dig = ((key >> 24) & 255) ^ 128
            plsc.addupdate_scatter(hist, [dig * 16 + lane], ones)
            return 0

        lax.fori_loop(0, _NV, keys_body, 0)

        def scan_body(i, carry):
            cum, bstar, below = carry
            h = hist[pl.ds(pl.multiple_of(i * 16, 8), 16)]
            s = jnp.sum(h)
            newcum = cum + s
            hit = (cum <= _K) & (newcum > _K)
            bstar = jnp.where(hit, i, bstar)
            below = jnp.where(hit, cum, below)
            return newcum, bstar, below

        _, bstar, below = lax.fori_loop(
            0, 256, scan_body,
            (jnp.int32(0), jnp.int32(0), jnp.int32(0)))

        def compact_body(j, offv):
            off8 = pl.multiple_of(j * 16, 8)
            key = keys[pl.ds(off8, 16)]
            dig = ((key >> 24) & 255) ^ 128
            m = dig == splat(bstar)
            mi = m.astype(jnp.int32)
            pos = offv + plsc.cumsum(mi) - mi
            plsc.store_scatter(cand, [pos], key ^ jnp.int32(_IMIN), mask=m)
            plsc.store_scatter(candidx, [pos], off8 + lane, mask=m)
            return offv + plsc.all_reduce_population_count(m)

        offv = lax.fori_loop(0, _NV, compact_body, zeros16)
        csize = jnp.max(offv)
        plsc.store_scatter(cand, [offv + lane], splat(jnp.int32(_IMAX)))
        plsc.store_scatter(candidx, [offv + lane], splat(jnp.int32(1 << 20)))
        nv = (csize + 15) // 16

        t_c = jnp.int32(_K) - below

        def count_lt(trial):
            def cbody(v, cnt):
                kv = cand[pl.ds(pl.multiple_of(v * 16, 8), 16)]
                return cnt + jnp.sum((kv < splat(trial)).astype(jnp.int32))
            return lax.fori_loop(0, nv, cbody, jnp.int32(0))

        lo = lax.shift_left(bstar, 24)
        for b in range(23, -1, -1):
            trial = lo | jnp.int32(1 << b)
            lo = jnp.where(count_lt(trial) <= t_c, trial, lo)
        uvkey = lo
        t = t_c - count_lt(uvkey)

        def count_idx(trial):
            def cbody(v, cnt):
                off8 = pl.multiple_of(v * 16, 8)
                kv = cand[pl.ds(off8, 16)]
                iv = candidx[pl.ds(off8, 16)]
                m = (kv == splat(uvkey)) & (iv < splat(trial))
                return cnt + jnp.sum(m.astype(jnp.int32))
            return lax.fori_loop(0, nv, cbody, jnp.int32(0))

        ilo = jnp.int32(0)
        for b in range(12, -1, -1):
            trial = ilo | jnp.int32(1 << b)
            ilo = jnp.where(count_idx(trial) <= t, trial, ilo)

        vkey = uvkey ^ jnp.int32(_IMIN)
        vbits = vkey ^ ((vkey >> 31) & jnp.int32(0x7FFFFFFF))
        vvec = lax.bitcast_convert_type(splat(vbits), jnp.float32)
        plsc.store_scatter(valbuf, [splat(r)], vvec, mask=lane0)
        plsc.store_scatter(idxbuf, [splat(r)], splat(ilo), mask=lane0)
        return 0

    lax.fori_loop(0, _RPW, row_body, 0)
    base = wid * _RPW
    pltpu.sync_copy(valbuf, val_hbm.at[pl.ds(base, _RPW)])
    pltpu.sync_copy(idxbuf, idx_hbm.at[pl.ds(base, _RPW)])


def _sc_call(x):
    mesh = plsc.VectorSubcoreMesh(core_axis_name="c", subcore_axis_name="s",
                                  num_cores=2, num_subcores=16)
    f = pl.kernel(
        _sc_body,
        out_type=[
            jax.ShapeDtypeStruct((_B - _SPLIT,), jnp.float32),
            jax.ShapeDtypeStruct((_B - _SPLIT,), jnp.int32),
        ],
        mesh=mesh,
        compiler_params=pltpu.CompilerParams(needs_layout_passes=False),
        scratch_types=[
            pltpu.VMEM((_N,), jnp.float32),
            pltpu.VMEM((_N,), jnp.int32),
            pltpu.VMEM((4096,), jnp.int32),
            pltpu.VMEM((_N + 16,), jnp.int32),
            pltpu.VMEM((_N + 16,), jnp.int32),
            pltpu.VMEM((_RPW,), jnp.float32),
            pltpu.VMEM((_RPW,), jnp.int32),
        ],
    )
    return f(x)


@jax.jit
def kernel(x):
    tv, ti = _tc_call(x)
    sv, si = _sc_call(x)
    return (jnp.concatenate([tv, sv]), jnp.concatenate([ti, si]))


# speedup vs baseline: 5.4894x; 1.0002x over previous
import jax
import jax.numpy as jnp
from jax import lax
from jax.experimental import pallas as pl
from jax.experimental.pallas import tpu as pltpu
from jax.experimental.pallas import tpu_sc as plsc

_B, _N = 4096, 8192
_K = (_N - 1) // 2
_SPLIT = 3584
_BR = 256
_NW = 32
_RPW = (_B - _SPLIT) // _NW
_NV = _N // 16
_IMIN = -(2 ** 31)
_IMAX = 2 ** 31 - 1


def _tc_block(x_ref, val_ref, idx_ref):
    x = x_ref[...]
    x = jnp.where(x == 0.0, jnp.float32(0.0), x)
    x = jnp.where(jnp.isnan(x), jnp.float32(jnp.nan), x)
    bits = jax.lax.bitcast_convert_type(x, jnp.int32)
    skey = bits ^ ((bits >> 31) & 0x7FFFFFFF)

    hi = (skey >> 16).astype(jnp.int16)
    lo = ((skey & 0xFFFF) - 32768).astype(jnp.int16)
    kk = jnp.int32(_K)

    def cnt16(mask16):
        m = mask16.astype(jnp.int16)
        while m.shape[1] > 512:
            h = m.shape[1] // 2
            m = m[:, :h] + m[:, h:]
        return jnp.sum(m.astype(jnp.int32), axis=1, keepdims=True)

    hu = jnp.zeros((_BR, 1), jnp.int32)
    for b in range(15, -1, -1):
        trial = ((hu | (1 << b)) - 32768).astype(jnp.int16)
        hu = jnp.where(cnt16(hi < trial) <= kk, hu | (1 << b), hu)
    hs = (hu - 32768).astype(jnp.int16)
    c_hi = cnt16(hi < hs)

    w2 = jnp.where(hi == hs, lo, jnp.int16(32767))
    lu = jnp.zeros((_BR, 1), jnp.int32)
    for b in range(15, -1, -1):
        trial = ((lu | (1 << b)) - 32768).astype(jnp.int16)
        lu = jnp.where(c_hi + cnt16(w2 < trial) <= kk, lu | (1 << b), lu)
    ls = (lu - 32768).astype(jnp.int16)
    t = kk - (c_hi + cnt16(w2 < ls))

    iot = jax.lax.broadcasted_iota(jnp.int32, (_BR, _N), 1).astype(jnp.int16)
    w3 = jnp.where((hi == hs) & (lo == ls), iot, jnp.int16(_N))
    ilo = jnp.zeros((_BR, 1), jnp.int32)
    for b in range(12, -1, -1):
        trial = (ilo | (1 << b)).astype(jnp.int16)
        ilo = jnp.where(cnt16(w3 < trial) <= t, ilo | (1 << b), ilo)

    vkey = (hs.astype(jnp.int32) << 16) | lu
    vbits = vkey ^ ((vkey >> 31) & 0x7FFFFFFF)
    val_ref[...] = jax.lax.bitcast_convert_type(vbits, jnp.float32)
    idx_ref[...] = ilo


def _tc_call(x):
    values, idx = pl.pallas_call(
        _tc_block,
        grid=(_SPLIT // _BR,),
        in_specs=[pl.BlockSpec((_BR, _N), lambda i: (i, 0))],
        out_specs=[
            pl.BlockSpec((_BR, 1), lambda i: (i, 0)),
            pl.BlockSpec((_BR, 1), lambda i: (i, 0)),
        ],
        out_shape=[
            jax.ShapeDtypeStruct((_SPLIT, 1), jnp.float32),
            jax.ShapeDtypeStruct((_SPLIT, 1), jnp.int32),
        ],
    )(x)
    return values[:, 0], idx[:, 0]


def _sc_body(x_hbm, val_hbm, idx_hbm, xbuf, keys, hist, cand, candidx,
             valbuf, idxbuf):
    nc = 2
    wid = lax.axis_index("s") * nc + lax.axis_index("c")
    lane = lax.iota(jnp.int32, 16)
    ones = jnp.ones((16,), jnp.int32)
    zeros16 = jnp.zeros((16,), jnp.int32)
    lane0 = lane == 0

    def splat(s):
        return jnp.broadcast_to(s, (16,))

    def row_body(r, _):
        g = _SPLIT + wid * _RPW + r
        pltpu.sync_copy(x_hbm.at[g], xbuf)

        def zero_body(i, _c):
            for u in range(4):
                hist[pl.ds(pl.multiple_of(i * 64 + u * 16, 8), 16)] = zeros16
            return 0

        lax.fori_loop(0, 64, zero_body, 0)

        def keys_body(j, _c):
            for u in range(4):
                off = pl.multiple_of(j * 64 + u * 16, 8)
                v = xbuf[pl.ds(off, 16)]
                v = jnp.where(v == 0.0, jnp.float32(0.0), v)
                v = jnp.where(v != v, jnp.float32(jnp.nan), v)
                bits = lax.bitcast_convert_type(v, jnp.int32)
                key = bits ^ ((bits >> 31) & jnp.int32(0x7FFFFFFF))
                keys[pl.ds(off, 16)] = key
                dig = ((key >> 24) & 255) ^ 128
                plsc.addupdate_scatter(hist, [dig * 16 + lane], ones)
            return 0

        lax.fori_loop(0, _NV // 4, keys_body, 0)

        def scan_body(i, carry):
            cum, bstar, below = carry
            for u in range(4):
                h = hist[pl.ds(pl.multiple_of(i * 64 + u * 16, 8), 16)]
                s = jnp.sum(h)
                newcum = cum + s
                hit = (cum <= _K) & (newcum > _K)
                bstar = jnp.where(hit, i * 4 + u, bstar)
                below = jnp.where(hit, cum, below)
                cum = newcum
            return cum, bstar, below

        _, bstar, below = lax.fori_loop(
            0, 64, scan_body,
            (jnp.int32(0), jnp.int32(0), jnp.int32(0)))

        def compact_body(j, offv):
            for u in range(4):
                off8 = pl.multiple_of(j * 64 + u * 16, 8)
                key = keys[pl.ds(off8, 16)]
                dig = ((key >> 24) & 255) ^ 128
                m = dig == splat(bstar)
                mi = m.astype(jnp.int32)
                pos = offv + plsc.cumsum(mi) - mi
                plsc.store_scatter(cand, [pos], key ^ jnp.int32(_IMIN),
                                   mask=m)
                plsc.store_scatter(candidx, [pos], off8 + lane, mask=m)
                offv = offv + plsc.all_reduce_population_count(m)
            return offv

        offv = lax.fori_loop(0, _NV // 4, compact_body, zeros16)
        csize = jnp.max(offv)
        plsc.store_scatter(cand, [offv + lane], splat(jnp.int32(_IMAX)))
        plsc.store_scatter(cand, [offv + 16 + lane], splat(jnp.int32(_IMAX)))
        plsc.store_scatter(candidx, [offv + lane], splat(jnp.int32(1 << 20)))
        plsc.store_scatter(candidx, [offv + 16 + lane],
                           splat(jnp.int32(1 << 20)))
        nv2 = (csize + 31) // 32

        t_c = jnp.int32(_K) - below

        def count_lt(trial):
            def cbody(v, cnt):
                for u in range(2):
                    kv = cand[pl.ds(pl.multiple_of(v * 32 + u * 16, 8), 16)]
                    cnt = cnt + jnp.sum(
                        (kv < splat(trial)).astype(jnp.int32))
                return cnt
            return lax.fori_loop(0, nv2, cbody, jnp.int32(0))

        lo = lax.shift_left(bstar, 24)
        for b in range(23, -1, -1):
            trial = lo | jnp.int32(1 << b)
            lo = jnp.where(count_lt(trial) <= t_c, trial, lo)
        uvkey = lo
        t = t_c - count_lt(uvkey)

        def count_idx(trial):
            def cbody(v, cnt):
                for u in range(2):
                    off8 = pl.multiple_of(v * 32 + u * 16, 8)
                    kv = cand[pl.ds(off8, 16)]
                    iv = candidx[pl.ds(off8, 16)]
                    m = (kv == splat(uvkey)) & (iv < splat(trial))
                    cnt = cnt + jnp.sum(m.astype(jnp.int32))
                return cnt
            return lax.fori_loop(0, nv2, cbody, jnp.int32(0))

        ilo = jnp.int32(0)
        for b in range(12, -1, -1):
            trial = ilo | jnp.int32(1 << b)
            ilo = jnp.where(count_idx(trial) <= t, trial, ilo)

        vkey = uvkey ^ jnp.int32(_IMIN)
        vbits = vkey ^ ((vkey >> 31) & jnp.int32(0x7FFFFFFF))
        vvec = lax.bitcast_convert_type(splat(vbits), jnp.float32)
        plsc.store_scatter(valbuf, [splat(r)], vvec, mask=lane0)
        plsc.store_scatter(idxbuf, [splat(r)], splat(ilo), mask=lane0)
        return 0

    lax.fori_loop(0, _RPW, row_body, 0)
    base = wid * _RPW
    pltpu.sync_copy(valbuf, val_hbm.at[pl.ds(base, _RPW)])
    pltpu.sync_copy(idxbuf, idx_hbm.at[pl.ds(base, _RPW)])


def _sc_call(x):
    mesh = plsc.VectorSubcoreMesh(core_axis_name="c", subcore_axis_name="s",
                                  num_cores=2, num_subcores=16)
    f = pl.kernel(
        _sc_body,
        out_type=[
            jax.ShapeDtypeStruct((_B - _SPLIT,), jnp.float32),
            jax.ShapeDtypeStruct((_B - _SPLIT,), jnp.int32),
        ],
        mesh=mesh,
        compiler_params=pltpu.CompilerParams(needs_layout_passes=False),
        scratch_types=[
            pltpu.VMEM((_N,), jnp.float32),
            pltpu.VMEM((_N,), jnp.int32),
            pltpu.VMEM((4096,), jnp.int32),
            pltpu.VMEM((_N + 32,), jnp.int32),
            pltpu.VMEM((_N + 32,), jnp.int32),
            pltpu.VMEM((_RPW,), jnp.float32),
            pltpu.VMEM((_RPW,), jnp.int32),
        ],
    )
    return f(x)


@jax.jit
def kernel(x):
    tv, ti = _tc_call(x)
    sv, si = _sc_call(x)
    return (jnp.concatenate([tv, sv]), jnp.concatenate([ti, si]))
